# Initial kernel scaffold; baseline (speedup 1.0000x reference)
import jax
import jax.numpy as jnp
from jax.experimental import pallas as pl


def kernel(x, edge_index, W1, b1, W2, b2, W3, b3, W4, b4):
    raise NotImplementedError("write your pallas kernel here")


# speedup vs baseline: 7.1731x; 7.1731x over previous
import functools

import jax
import jax.numpy as jnp
from jax import lax
from jax.experimental import pallas as pl
from jax.experimental.pallas import tpu as pltpu
from jax.experimental.pallas import tpu_sc as plsc

N = 10000
E = 320000
NC = 2
NS = 16
NW = NC * NS
B = 80
C = E // NW // B
C2 = E // NS // B
NP_ = 10240
RPT = NP_ // NS
ZB = 128
DH = 64


def _fill(ref, rows, cols, value):
    k = cols // 16

    def body(i, carry):
        r = i // k
        q = (i % k) * 16
        ref[r, pl.ds(q, 16)] = jnp.full((16,), value, jnp.float32)
        return carry

    lax.fori_loop(0, rows * k, body, 0)


def _mesh():
    return plsc.VectorSubcoreMesh(core_axis_name="c", subcore_axis_name="s")


def _make_degree_kernel():
    @functools.partial(
        pl.kernel,
        mesh=_mesh(),
        compiler_params=pltpu.CompilerParams(use_tc_tiling_on_sc=False),
        out_type=(
            jax.ShapeDtypeStruct((NC * NP_, 16), jnp.float32),
            jax.ShapeDtypeStruct((NC * NP_, 16), jnp.float32),
        ),
        scratch_types=[
            pltpu.VMEM((C, B), jnp.int32),
            pltpu.VMEM((C, B), jnp.int32),
            pltpu.VMEM((B, 16), jnp.float32),
            pltpu.VMEM((ZB, 16), jnp.float32),
            pltpu.VMEM_SHARED((NP_, 16), jnp.float32),
            pltpu.VMEM_SHARED((NP_, 16), jnp.float32),
        ],
    )
    def deg_kernel(srcs, dsts, out_o, out_i, src_v, dst_v, ones_v, zeros_v,
                   acc_o, acc_i):
        c = lax.axis_index("c")
        s = lax.axis_index("s")
        wid = s * NC + c
        pltpu.sync_copy(srcs.at[wid], src_v)
        pltpu.sync_copy(dsts.at[wid], dst_v)
        _fill(ones_v, B, 16, 1.0)
        _fill(zeros_v, ZB, 16, 0.0)
        base = s * RPT
        for t in range(RPT // ZB):
            pltpu.sync_copy(zeros_v, acc_o.at[pl.ds(base + t * ZB, ZB)])
            pltpu.sync_copy(zeros_v, acc_i.at[pl.ds(base + t * ZB, ZB)])
        plsc.subcore_barrier()

        def body(j, carry):
            pltpu.sync_copy(ones_v, acc_o.at[src_v.at[j]], add=True)
            pltpu.sync_copy(ones_v, acc_i.at[dst_v.at[j]], add=True)
            return carry

        lax.fori_loop(0, C, body, 0)
        plsc.subcore_barrier()
        pltpu.sync_copy(acc_o.at[pl.ds(base, RPT)],
                        out_o.at[pl.ds(c * NP_ + base, RPT)])
        pltpu.sync_copy(acc_i.at[pl.ds(base, RPT)],
                        out_i.at[pl.ds(c * NP_ + base, RPT)])

    return deg_kernel


def _make_agg_kernel(D):
    @functools.partial(
        pl.kernel,
        mesh=_mesh(),
        compiler_params=pltpu.CompilerParams(use_tc_tiling_on_sc=False),
        out_type=jax.ShapeDtypeStruct((NC * NP_, D), jnp.float32),
        scratch_types=[
            pltpu.VMEM((C, B), jnp.int32),
            pltpu.VMEM((C, B), jnp.int32),
            pltpu.VMEM((B, D), jnp.float32),
            pltpu.VMEM((ZB, D), jnp.float32),
            pltpu.VMEM_SHARED((NP_, D), jnp.float32),
            pltpu.SemaphoreType.DMA,
        ],
    )
    def agg_kernel(table, srcs, dsts, out, src_v, dst_v, rows_v, zeros_v,
                   acc, sem):
        c = lax.axis_index("c")
        s = lax.axis_index("s")
        wid = s * NC + c
        pltpu.sync_copy(srcs.at[wid], src_v)
        pltpu.sync_copy(dsts.at[wid], dst_v)
        _fill(zeros_v, ZB, D, 0.0)
        base = s * RPT
        for t in range(RPT // ZB):
            pltpu.sync_copy(zeros_v, acc.at[pl.ds(base + t * ZB, ZB)])
        plsc.subcore_barrier()

        def body(j, carry):
            pltpu.async_copy(table.at[src_v.at[j]], rows_v, sem).wait()
            pltpu.sync_copy(rows_v, acc.at[dst_v.at[j]], add=True)
            return carry

        lax.fori_loop(0, C, body, 0)
        plsc.subcore_barrier()
        pltpu.sync_copy(acc.at[pl.ds(base, RPT)],
                        out.at[pl.ds(c * NP_ + base, RPT)])

    return agg_kernel


def _make_agg_split_kernel():
    @functools.partial(
        pl.kernel,
        mesh=_mesh(),
        compiler_params=pltpu.CompilerParams(use_tc_tiling_on_sc=False),
        out_type=jax.ShapeDtypeStruct((NC, NP_, DH), jnp.float32),
        scratch_types=[
            pltpu.VMEM((C2, B), jnp.int32),
            pltpu.VMEM((C2, B), jnp.int32),
            pltpu.VMEM((B, DH), jnp.float32),
            pltpu.VMEM((ZB, DH), jnp.float32),
            pltpu.VMEM_SHARED((NP_, DH), jnp.float32),
            pltpu.SemaphoreType.DMA,
        ],
    )
    def agg_kernel(table2, srcs, dsts, out, src_v, dst_v, rows_v, zeros_v,
                   acc, sem):
        c = lax.axis_index("c")
        s = lax.axis_index("s")
        pltpu.sync_copy(srcs.at[s], src_v)
        pltpu.sync_copy(dsts.at[s], dst_v)
        _fill(zeros_v, ZB, DH, 0.0)
        base = s * RPT
        for t in range(RPT // ZB):
            pltpu.sync_copy(zeros_v, acc.at[pl.ds(base + t * ZB, ZB)])
        plsc.subcore_barrier()

        def body(j, carry):
            pltpu.async_copy(table2.at[c].at[src_v.at[j]], rows_v, sem).wait()
            pltpu.sync_copy(rows_v, acc.at[dst_v.at[j]], add=True)
            return carry

        lax.fori_loop(0, C2, body, 0)
        plsc.subcore_barrier()
        pltpu.sync_copy(acc.at[pl.ds(base, RPT)],
                        out.at[c].at[pl.ds(base, RPT)])

    return agg_kernel


_DEG = _make_degree_kernel()
_AGG = {d: _make_agg_kernel(d) for d in (32, 16)}
_AGG_SPLIT = _make_agg_split_kernel()


def _norms_body(do_ref, di_ref, x_ref, ns_ref, nd_ref, t1_ref):
    do = do_ref[...]
    di = di_ref[...]
    deg_o = jnp.sum(do[:N] + do[NP_:NP_ + N], axis=1, keepdims=True) * (1.0 / 16.0)
    deg_i = jnp.sum(di[:N] + di[NP_:NP_ + N], axis=1, keepdims=True) * (1.0 / 16.0)
    ns = lax.rsqrt(jnp.maximum(deg_o, 1.0))
    nd = lax.rsqrt(jnp.maximum(deg_i, 1.0))
    ns_ref[...] = ns
    nd_ref[...] = nd
    xs = x_ref[...] * ns
    t1_ref[0] = xs[:, :DH]
    t1_ref[1] = xs[:, DH:]


def _stage2_body(p_ref, nd_ref, ns_ref, w1_ref, b1_ref, w2_ref, o_ref):
    p = p_ref[...]
    m = jnp.concatenate([p[0, :N], p[1, :N]], axis=1) * nd_ref[...]
    h = jnp.dot(m, w1_ref[...], preferred_element_type=jnp.float32)
    h = jnp.maximum(h + b1_ref[...][None, :], 0.0)
    o_ref[...] = jnp.dot(h * ns_ref[...], w2_ref[...],
                         preferred_element_type=jnp.float32)


def _stage3_body(p_ref, nd_ref, ns_ref, b2_ref, w3_ref, o_ref):
    p = p_ref[...]
    m = (p[:N] + p[NP_:NP_ + N]) * nd_ref[...]
    h = jnp.maximum(m + b2_ref[...][None, :], 0.0)
    o_ref[...] = jnp.dot(h * ns_ref[...], w3_ref[...],
                         preferred_element_type=jnp.float32)


def _stage4_body(p_ref, nd_ref, ns_ref, b3_ref, o_ref):
    p = p_ref[...]
    m = (p[:N] + p[NP_:NP_ + N]) * nd_ref[...]
    h = jnp.maximum(m + b3_ref[...][None, :], 0.0)
    o_ref[...] = h * ns_ref[...]


def _stage5_body(p_ref, nd_ref, w4_ref, b4_ref, o_ref):
    p = p_ref[...]
    m = (p[:N] + p[NP_:NP_ + N]) * nd_ref[...]
    o_ref[...] = (jnp.dot(m, w4_ref[...], preferred_element_type=jnp.float32)
                  + b4_ref[...][None, :])


def _f32(*shape):
    return jax.ShapeDtypeStruct(shape, jnp.float32)


def kernel(x, edge_index, W1, b1, W2, b2, W3, b3, W4, b4):
    src_r = edge_index[0].reshape(NW, C, B)
    dst_r = edge_index[1].reshape(NW, C, B)
    src_r16 = edge_index[0].reshape(NS, C2, B)
    dst_r16 = edge_index[1].reshape(NS, C2, B)

    deg_o_p, deg_i_p = _DEG(src_r, dst_r)
    ns, nd, t1 = pl.pallas_call(
        _norms_body,
        out_shape=(_f32(N, 1), _f32(N, 1), _f32(NC, N, DH)),
    )(deg_o_p, deg_i_p, x)

    p1 = _AGG_SPLIT(t1, src_r16, dst_r16)
    t2 = pl.pallas_call(_stage2_body, out_shape=_f32(N, 32))(
        p1, nd, ns, W1, b1, W2)

    p2 = _AGG[32](t2, src_r, dst_r)
    t3 = pl.pallas_call(_stage3_body, out_shape=_f32(N, 16))(
        p2, nd, ns, b2, W3)

    p3 = _AGG[16](t3, src_r, dst_r)
    t4 = pl.pallas_call(_stage4_body, out_shape=_f32(N, 16))(
        p3, nd, ns, b3)

    p4 = _AGG[16](t4, src_r, dst_r)
    out = pl.pallas_call(_stage5_body, out_shape=_f32(N, 40))(
        p4, nd, W4, b4)
    return out


# speedup vs baseline: 8.8839x; 1.2385x over previous
import functools

import jax
import jax.numpy as jnp
from jax import lax
from jax.experimental import pallas as pl
from jax.experimental.pallas import tpu as pltpu
from jax.experimental.pallas import tpu_sc as plsc

N = 10000
E = 320000
NC = 2
NS = 16
NW = NC * NS
EW = E // NW
ET = E // NS
BP = 128
CP = -(-EW // BP)
CP2 = -(-ET // BP)
NP_ = 10240
RPT = NP_ // NS
ZB = 128
DH = 64


def _fill(ref, rows, cols, value):
    k = cols // 16

    def body(i, carry):
        r = i // k
        q = (i % k) * 16
        ref[r, pl.ds(q, 16)] = jnp.full((16,), value, jnp.float32)
        return carry

    lax.fori_loop(0, rows * k, body, 0)


def _mesh():
    return plsc.VectorSubcoreMesh(core_axis_name="c", subcore_axis_name="s")


def _zero_acc(zeros_v, acc, base):
    for t in range(RPT // ZB):
        pltpu.sync_copy(zeros_v, acc.at[pl.ds(base + t * ZB, ZB)])


def _gather_scatter_loop(table, src_v, dst_v, rows2_v, acc, sem, nchunks):
    pltpu.async_copy(table.at[src_v.at[0]], rows2_v.at[0], sem)

    def body(j, carry):
        b = j % 2
        pltpu.make_async_copy(table.at[src_v.at[j]], rows2_v.at[b], sem).wait()

        @pl.when(j + 1 < nchunks)
        def _():
            pltpu.async_copy(table.at[src_v.at[j + 1]], rows2_v.at[1 - b], sem)

        pltpu.sync_copy(rows2_v.at[b], acc.at[dst_v.at[j]], add=True)
        return carry

    lax.fori_loop(0, nchunks, body, 0)


def _make_degree_kernel():
    @functools.partial(
        pl.kernel,
        mesh=_mesh(),
        compiler_params=pltpu.CompilerParams(use_tc_tiling_on_sc=False),
        out_type=(
            jax.ShapeDtypeStruct((NC * NP_, 16), jnp.float32),
            jax.ShapeDtypeStruct((NC * NP_, 16), jnp.float32),
        ),
        scratch_types=[
            pltpu.VMEM((CP, BP), jnp.int32),
            pltpu.VMEM((CP, BP), jnp.int32),
            pltpu.VMEM((BP, 16), jnp.float32),
            pltpu.VMEM((ZB, 16), jnp.float32),
            pltpu.VMEM_SHARED((NP_, 16), jnp.float32),
            pltpu.VMEM_SHARED((NP_, 16), jnp.float32),
        ],
    )
    def deg_kernel(srcs, dsts, out_o, out_i, src_v, dst_v, ones_v, zeros_v,
                   acc_o, acc_i):
        c = lax.axis_index("c")
        s = lax.axis_index("s")
        wid = s * NC + c
        pltpu.sync_copy(srcs.at[wid], src_v)
        pltpu.sync_copy(dsts.at[wid], dst_v)
        _fill(ones_v, BP, 16, 1.0)
        _fill(zeros_v, ZB, 16, 0.0)
        base = s * RPT
        _zero_acc(zeros_v, acc_o, base)
        _zero_acc(zeros_v, acc_i, base)
        plsc.subcore_barrier()

        def body(j, carry):
            pltpu.sync_copy(ones_v, acc_o.at[src_v.at[j]], add=True)
            pltpu.sync_copy(ones_v, acc_i.at[dst_v.at[j]], add=True)
            return carry

        lax.fori_loop(0, CP, body, 0)
        plsc.subcore_barrier()
        pltpu.sync_copy(acc_o.at[pl.ds(base, RPT)],
                        out_o.at[pl.ds(c * NP_ + base, RPT)])
        pltpu.sync_copy(acc_i.at[pl.ds(base, RPT)],
                        out_i.at[pl.ds(c * NP_ + base, RPT)])

    return deg_kernel


def _make_agg_kernel(D):
    @functools.partial(
        pl.kernel,
        mesh=_mesh(),
        compiler_params=pltpu.CompilerParams(use_tc_tiling_on_sc=False),
        out_type=jax.ShapeDtypeStruct((NC * NP_, D), jnp.float32),
        scratch_types=[
            pltpu.VMEM((CP, BP), jnp.int32),
            pltpu.VMEM((CP, BP), jnp.int32),
            pltpu.VMEM((2, BP, D), jnp.float32),
            pltpu.VMEM((ZB, D), jnp.float32),
            pltpu.VMEM_SHARED((NP_, D), jnp.float32),
            pltpu.SemaphoreType.DMA,
        ],
    )
    def agg_kernel(table, srcs, dsts, out, src_v, dst_v, rows2_v, zeros_v,
                   acc, sem):
        c = lax.axis_index("c")
        s = lax.axis_index("s")
        wid = s * NC + c
        pltpu.sync_copy(srcs.at[wid], src_v)
        pltpu.sync_copy(dsts.at[wid], dst_v)
        _fill(zeros_v, ZB, D, 0.0)
        base = s * RPT
        _zero_acc(zeros_v, acc, base)
        plsc.subcore_barrier()
        _gather_scatter_loop(table, src_v, dst_v, rows2_v, acc, sem, CP)
        plsc.subcore_barrier()
        pltpu.sync_copy(acc.at[pl.ds(base, RPT)],
                        out.at[pl.ds(c * NP_ + base, RPT)])

    return agg_kernel


def _make_agg_split_kernel():
    @functools.partial(
        pl.kernel,
        mesh=_mesh(),
        compiler_params=pltpu.CompilerParams(use_tc_tiling_on_sc=False),
        out_type=jax.ShapeDtypeStruct((NC, NP_, DH), jnp.float32),
        scratch_types=[
            pltpu.VMEM((CP2, BP), jnp.int32),
            pltpu.VMEM((CP2, BP), jnp.int32),
            pltpu.VMEM((2, BP, DH), jnp.float32),
            pltpu.VMEM((ZB, DH), jnp.float32),
            pltpu.VMEM_SHARED((NP_, DH), jnp.float32),
            pltpu.SemaphoreType.DMA,
        ],
    )
    def agg_kernel(table2, srcs, dsts, out, src_v, dst_v, rows2_v, zeros_v,
                   acc, sem):
        c = lax.axis_index("c")
        s = lax.axis_index("s")
        pltpu.sync_copy(srcs.at[s], src_v)
        pltpu.sync_copy(dsts.at[s], dst_v)
        _fill(zeros_v, ZB, DH, 0.0)
        base = s * RPT
        _zero_acc(zeros_v, acc, base)
        plsc.subcore_barrier()
        _gather_scatter_loop(table2.at[c], src_v, dst_v, rows2_v, acc, sem,
                             CP2)
        plsc.subcore_barrier()
        pltpu.sync_copy(acc.at[pl.ds(base, RPT)],
                        out.at[c].at[pl.ds(base, RPT)])

    return agg_kernel


_DEG = _make_degree_kernel()
_AGG = {d: _make_agg_kernel(d) for d in (32, 16)}
_AGG_SPLIT = _make_agg_split_kernel()


def _norms_body(do_ref, di_ref, x_ref, ns_ref, nd_ref, t1_ref):
    do = do_ref[...]
    di = di_ref[...]
    deg_o = jnp.sum(do[:N] + do[NP_:NP_ + N], axis=1, keepdims=True) * (1.0 / 16.0)
    deg_i = jnp.sum(di[:N] + di[NP_:NP_ + N], axis=1, keepdims=True) * (1.0 / 16.0)
    ns = lax.rsqrt(jnp.maximum(deg_o, 1.0))
    nd = lax.rsqrt(jnp.maximum(deg_i, 1.0))
    ns_ref[...] = ns
    nd_ref[...] = nd
    xs = x_ref[...] * ns
    t1_ref[0] = xs[:, :DH]
    t1_ref[1] = xs[:, DH:]


def _stage2_body(p_ref, nd_ref, ns_ref, w1_ref, b1_ref, w2_ref, o_ref):
    p = p_ref[...]
    m = jnp.concatenate([p[0, :N], p[1, :N]], axis=1) * nd_ref[...]
    h = jnp.dot(m, w1_ref[...], preferred_element_type=jnp.float32)
    h = jnp.maximum(h + b1_ref[...][None, :], 0.0)
    o_ref[...] = jnp.dot(h * ns_ref[...], w2_ref[...],
                         preferred_element_type=jnp.float32)


def _stage3_body(p_ref, nd_ref, ns_ref, b2_ref, w3_ref, o_ref):
    p = p_ref[...]
    m = (p[:N] + p[NP_:NP_ + N]) * nd_ref[...]
    h = jnp.maximum(m + b2_ref[...][None, :], 0.0)
    o_ref[...] = jnp.dot(h * ns_ref[...], w3_ref[...],
                         preferred_element_type=jnp.float32)


def _stage4_body(p_ref, nd_ref, ns_ref, b3_ref, o_ref):
    p = p_ref[...]
    m = (p[:N] + p[NP_:NP_ + N]) * nd_ref[...]
    h = jnp.maximum(m + b3_ref[...][None, :], 0.0)
    o_ref[...] = h * ns_ref[...]


def _stage5_body(p_ref, nd_ref, w4_ref, b4_ref, o_ref):
    p = p_ref[...]
    m = (p[:N] + p[NP_:NP_ + N]) * nd_ref[...]
    o_ref[...] = (jnp.dot(m, w4_ref[...], preferred_element_type=jnp.float32)
                  + b4_ref[...][None, :])


def _f32(*shape):
    return jax.ShapeDtypeStruct(shape, jnp.float32)


def _pad_idx(idx, nsplit, pad_value):
    per = E // nsplit
    chunks = -(-per // BP)
    padded = jnp.pad(idx.reshape(nsplit, per),
                     ((0, 0), (0, chunks * BP - per)),
                     constant_values=pad_value)
    return padded.reshape(nsplit, chunks, BP)


def kernel(x, edge_index, W1, b1, W2, b2, W3, b3, W4, b4):
    src_r = _pad_idx(edge_index[0], NW, 0)
    dst_r = _pad_idx(edge_index[1], NW, N)
    src_r16 = _pad_idx(edge_index[0], NS, 0)
    dst_r16 = _pad_idx(edge_index[1], NS, N)

    deg_o_p, deg_i_p = _DEG(src_r, dst_r)
    ns, nd, t1 = pl.pallas_call(
        _norms_body,
        out_shape=(_f32(N, 1), _f32(N, 1), _f32(NC, N, DH)),
    )(deg_o_p, deg_i_p, x)

    p1 = _AGG_SPLIT(t1, src_r16, dst_r16)
    t2 = pl.pallas_call(_stage2_body, out_shape=_f32(N, 32))(
        p1, nd, ns, W1, b1, W2)

    p2 = _AGG[32](t2, src_r, dst_r)
    t3 = pl.pallas_call(_stage3_body, out_shape=_f32(N, 16))(
        p2, nd, ns, b2, W3)

    p3 = _AGG[16](t3, src_r, dst_r)
    t4 = pl.pallas_call(_stage4_body, out_shape=_f32(N, 16))(
        p3, nd, ns, b3)

    p4 = _AGG[16](t4, src_r, dst_r)
    out = pl.pallas_call(_stage5_body, out_shape=_f32(N, 40))(
        p4, nd, W4, b4)
    return out


# speedup vs baseline: 9.1980x; 1.0354x over previous
import functools

import jax
import jax.numpy as jnp
from jax import lax
from jax.experimental import pallas as pl
from jax.experimental.pallas import tpu as pltpu
from jax.experimental.pallas import tpu_sc as plsc

N = 10000
E = 320000
NC = 2
NS = 16
NW = NC * NS
BP = 128
NP_ = 10240
RPT = NP_ // NS
ZB = 128
DH = 64

G16 = 10
G32 = 5
GSP = 2
GDEG = 20


def _nblocks(per_worker, g):
    return -(-per_worker // (g * BP))


def _fill(ref, rows, cols, value):
    k = cols // 16

    def body(i, carry):
        r = i // k
        q = (i % k) * 16
        ref[r, pl.ds(q, 16)] = jnp.full((16,), value, jnp.float32)
        return carry

    lax.fori_loop(0, rows * k, body, 0)


def _fill3(ref, a, b, cols, value):
    k = cols // 16

    def body(i, carry):
        r = i // (b * k)
        rem = i % (b * k)
        q = rem // k
        w = (rem % k) * 16
        ref[r, q, pl.ds(w, 16)] = jnp.full((16,), value, jnp.float32)
        return carry

    lax.fori_loop(0, a * b * k, body, 0)


def _mesh():
    return plsc.VectorSubcoreMesh(core_axis_name="c", subcore_axis_name="s")


def _zero_acc(zeros_v, acc, base):
    for t in range(RPT // ZB):
        pltpu.sync_copy(zeros_v, acc.at[pl.ds(base + t * ZB, ZB)])


def _gather_scatter_loop(table, src_v, dst_v, rows2_v, acc, sem_g, sem_s,
                         nblk):
    pltpu.async_copy(table.at[src_v.at[0]], rows2_v.at[0], sem_g)

    def body(j, carry):
        b = j % 2
        pltpu.make_async_copy(table.at[src_v.at[j]], rows2_v.at[b],
                              sem_g).wait()
        pltpu.async_copy(rows2_v.at[b], acc.at[dst_v.at[j]], sem_s, add=True)

        @pl.when(j >= 1)
        def _():
            pltpu.make_async_copy(rows2_v.at[1 - b], acc.at[dst_v.at[j - 1]],
                                  sem_s).wait()

        @pl.when(j + 1 < nblk)
        def _():
            pltpu.async_copy(table.at[src_v.at[j + 1]], rows2_v.at[1 - b],
                             sem_g)

        return carry

    lax.fori_loop(0, nblk, body, 0)
    pltpu.make_async_copy(rows2_v.at[(nblk - 1) % 2],
                          acc.at[dst_v.at[nblk - 1]], sem_s).wait()


def _make_degree_kernel():
    nb = _nblocks(E // NW, GDEG)

    @functools.partial(
        pl.kernel,
        mesh=_mesh(),
        compiler_params=pltpu.CompilerParams(use_tc_tiling_on_sc=False),
        out_type=(
            jax.ShapeDtypeStruct((NC * NP_, 16), jnp.float32),
            jax.ShapeDtypeStruct((NC * NP_, 16), jnp.float32),
        ),
        scratch_types=[
            pltpu.VMEM((nb, GDEG * BP), jnp.int32),
            pltpu.VMEM((nb, GDEG * BP), jnp.int32),
            pltpu.VMEM((GDEG * BP, 16), jnp.float32),
            pltpu.VMEM((ZB, 16), jnp.float32),
            pltpu.VMEM_SHARED((NP_, 16), jnp.float32),
            pltpu.VMEM_SHARED((NP_, 16), jnp.float32),
            pltpu.SemaphoreType.DMA,
        ],
    )
    def deg_kernel(srcs, dsts, out_o, out_i, src_v, dst_v, ones_v, zeros_v,
                   acc_o, acc_i, sem):
        c = lax.axis_index("c")
        s = lax.axis_index("s")
        wid = s * NC + c
        pltpu.sync_copy(srcs.at[wid], src_v)
        pltpu.sync_copy(dsts.at[wid], dst_v)
        _fill(ones_v, GDEG * BP, 16, 1.0)
        _fill(zeros_v, ZB, 16, 0.0)
        base = s * RPT
        _zero_acc(zeros_v, acc_o, base)
        _zero_acc(zeros_v, acc_i, base)
        plsc.subcore_barrier()

        def body(j, carry):
            pltpu.async_copy(ones_v, acc_o.at[src_v.at[j]], sem, add=True)
            pltpu.async_copy(ones_v, acc_i.at[dst_v.at[j]], sem, add=True)
            return carry

        lax.fori_loop(0, nb, body, 0)

        def drain(j, carry):
            pltpu.make_async_copy(ones_v, acc_o.at[src_v.at[0]], sem).wait()
            return carry

        lax.fori_loop(0, 2 * nb, drain, 0)
        plsc.subcore_barrier()
        pltpu.sync_copy(acc_o.at[pl.ds(base, RPT)],
                        out_o.at[pl.ds(c * NP_ + base, RPT)])
        pltpu.sync_copy(acc_i.at[pl.ds(base, RPT)],
                        out_i.at[pl.ds(c * NP_ + base, RPT)])

    return deg_kernel


def _make_agg_kernel(D, g):
    nb = _nblocks(E // NW, g)

    @functools.partial(
        pl.kernel,
        mesh=_mesh(),
        compiler_params=pltpu.CompilerParams(use_tc_tiling_on_sc=False),
        out_type=jax.ShapeDtypeStruct((NC * NP_, D), jnp.float32),
        scratch_types=[
            pltpu.VMEM((nb, g * BP), jnp.int32),
            pltpu.VMEM((nb, g * BP), jnp.int32),
            pltpu.VMEM((2, g * BP, D), jnp.float32),
            pltpu.VMEM((ZB, D), jnp.float32),
            pltpu.VMEM_SHARED((NP_, D), jnp.float32),
            pltpu.SemaphoreType.DMA,
            pltpu.SemaphoreType.DMA,
        ],
    )
    def agg_kernel(table, srcs, dsts, out, src_v, dst_v, rows2_v, zeros_v,
                   acc, sem_g, sem_s):
        c = lax.axis_index("c")
        s = lax.axis_index("s")
        wid = s * NC + c
        pltpu.sync_copy(srcs.at[wid], src_v)
        pltpu.sync_copy(dsts.at[wid], dst_v)
        _fill(zeros_v, ZB, D, 0.0)
        base = s * RPT
        _zero_acc(zeros_v, acc, base)
        plsc.subcore_barrier()
        _gather_scatter_loop(table, src_v, dst_v, rows2_v, acc, sem_g, sem_s,
                             nb)
        plsc.subcore_barrier()
        pltpu.sync_copy(acc.at[pl.ds(base, RPT)],
                        out.at[pl.ds(c * NP_ + base, RPT)])

    return agg_kernel


def _make_agg_split_kernel():
    nb = _nblocks(E // NS, GSP)

    @functools.partial(
        pl.kernel,
        mesh=_mesh(),
        compiler_params=pltpu.CompilerParams(use_tc_tiling_on_sc=False),
        out_type=jax.ShapeDtypeStruct((NC, NP_, DH), jnp.float32),
        scratch_types=[
            pltpu.VMEM((nb, GSP * BP), jnp.int32),
            pltpu.VMEM((nb, GSP * BP), jnp.int32),
            pltpu.VMEM((2, GSP * BP, DH), jnp.float32),
            pltpu.VMEM((ZB, DH), jnp.float32),
            pltpu.VMEM_SHARED((NP_, DH), jnp.float32),
            pltpu.SemaphoreType.DMA,
            pltpu.SemaphoreType.DMA,
        ],
    )
    def agg_kernel(table2, srcs, dsts, out, src_v, dst_v, rows2_v, zeros_v,
                   acc, sem_g, sem_s):
        c = lax.axis_index("c")
        s = lax.axis_index("s")
        pltpu.sync_copy(srcs.at[s], src_v)
        pltpu.sync_copy(dsts.at[s], dst_v)
        _fill(zeros_v, ZB, DH, 0.0)
        base = s * RPT
        _zero_acc(zeros_v, acc, base)
        plsc.subcore_barrier()
        _gather_scatter_loop(table2.at[c], src_v, dst_v, rows2_v, acc, sem_g,
                             sem_s, nb)
        plsc.subcore_barrier()
        pltpu.sync_copy(acc.at[pl.ds(base, RPT)],
                        out.at[c].at[pl.ds(base, RPT)])

    return agg_kernel


_DEG = _make_degree_kernel()
_AGG = {32: _make_agg_kernel(32, G32), 16: _make_agg_kernel(16, G16)}
_AGG_SPLIT = _make_agg_split_kernel()


def _norms_body(do_ref, di_ref, x_ref, ns_ref, nd_ref, t1_ref):
    do = do_ref[...]
    di = di_ref[...]
    deg_o = jnp.sum(do[:N] + do[NP_:NP_ + N], axis=1, keepdims=True) * (1.0 / 16.0)
    deg_i = jnp.sum(di[:N] + di[NP_:NP_ + N], axis=1, keepdims=True) * (1.0 / 16.0)
    ns = lax.rsqrt(jnp.maximum(deg_o, 1.0))
    nd = lax.rsqrt(jnp.maximum(deg_i, 1.0))
    ns_ref[...] = ns
    nd_ref[...] = nd
    xs = x_ref[...] * ns
    t1_ref[0] = xs[:, :DH]
    t1_ref[1] = xs[:, DH:]


def _stage2_body(p_ref, nd_ref, ns_ref, w1_ref, b1_ref, w2_ref, o_ref):
    p = p_ref[...]
    m = jnp.concatenate([p[0, :N], p[1, :N]], axis=1) * nd_ref[...]
    h = jnp.dot(m, w1_ref[...], preferred_element_type=jnp.float32)
    h = jnp.maximum(h + b1_ref[...][None, :], 0.0)
    o_ref[...] = jnp.dot(h * ns_ref[...], w2_ref[...],
                         preferred_element_type=jnp.float32)


def _stage3_body(p_ref, nd_ref, ns_ref, b2_ref, w3_ref, o_ref):
    p = p_ref[...]
    m = (p[:N] + p[NP_:NP_ + N]) * nd_ref[...]
    h = jnp.maximum(m + b2_ref[...][None, :], 0.0)
    o_ref[...] = jnp.dot(h * ns_ref[...], w3_ref[...],
                         preferred_element_type=jnp.float32)


def _stage4_body(p_ref, nd_ref, ns_ref, b3_ref, o_ref):
    p = p_ref[...]
    m = (p[:N] + p[NP_:NP_ + N]) * nd_ref[...]
    h = jnp.maximum(m + b3_ref[...][None, :], 0.0)
    o_ref[...] = h * ns_ref[...]


def _stage5_body(p_ref, nd_ref, w4_ref, b4_ref, o_ref):
    p = p_ref[...]
    m = (p[:N] + p[NP_:NP_ + N]) * nd_ref[...]
    o_ref[...] = (jnp.dot(m, w4_ref[...], preferred_element_type=jnp.float32)
                  + b4_ref[...][None, :])


def _f32(*shape):
    return jax.ShapeDtypeStruct(shape, jnp.float32)


def _pad_idx(idx, nsplit, g, pad_value):
    per = E // nsplit
    nb = _nblocks(per, g)
    tot = nb * g * BP
    padded = jnp.pad(idx.reshape(nsplit, per), ((0, 0), (0, tot - per)),
                     constant_values=pad_value)
    return padded.reshape(nsplit, nb, g * BP)


def kernel(x, edge_index, W1, b1, W2, b2, W3, b3, W4, b4):
    src, dst = edge_index[0], edge_index[1]
    srcs = {g: _pad_idx(src, NW, g, 0) for g in (GDEG, G32, G16)}
    dsts = {g: _pad_idx(dst, NW, g, N) for g in (GDEG, G32, G16)}
    src_sp = _pad_idx(src, NS, GSP, 0)
    dst_sp = _pad_idx(dst, NS, GSP, N)

    deg_o_p, deg_i_p = _DEG(srcs[GDEG], dsts[GDEG])
    ns, nd, t1 = pl.pallas_call(
        _norms_body,
        out_shape=(_f32(N, 1), _f32(N, 1), _f32(NC, N, DH)),
    )(deg_o_p, deg_i_p, x)

    p1 = _AGG_SPLIT(t1, src_sp, dst_sp)
    t2 = pl.pallas_call(_stage2_body, out_shape=_f32(N, 32))(
        p1, nd, ns, W1, b1, W2)

    p2 = _AGG[32](t2, srcs[G32], dsts[G32])
    t3 = pl.pallas_call(_stage3_body, out_shape=_f32(N, 16))(
        p2, nd, ns, b2, W3)

    p3 = _AGG[16](t3, srcs[G16], dsts[G16])
    t4 = pl.pallas_call(_stage4_body, out_shape=_f32(N, 16))(
        p3, nd, ns, b3)

    p4 = _AGG[16](t4, srcs[G16], dsts[G16])
    out = pl.pallas_call(_stage5_body, out_shape=_f32(N, 40))(
        p4, nd, W4, b4)
    return out
